# baseline (speedup 1.0000x reference)
import jax
import jax.numpy as jnp
from jax import lax
from jax.experimental import pallas as pl
from jax.experimental.pallas import tpu as pltpu
from jax.experimental.pallas import tpu_sc as plsc

_L = 512
_FN = 1024
_NPH = 8


def _build_f8_sc(table):
    max_len, d_model = table.shape

    info = plsc.get_sparse_core_info()
    num_workers = info.num_cores * info.num_subcores
    rows_per_worker = _NPH * _FN // num_workers
    chunk = 128

    mesh = plsc.VectorSubcoreMesh(core_axis_name="c", subcore_axis_name="s")

    @pl.kernel(
        out_type=jax.ShapeDtypeStruct((_NPH, _FN, d_model), jnp.float32),
        mesh=mesh,
        scratch_types=[
            pltpu.VMEM((chunk,), jnp.int32),
            pltpu.VMEM((chunk, d_model), jnp.float32),
            pltpu.SemaphoreType.DMA,
        ],
    )
    def k(table_hbm, f_hbm, idx_v, rows_v, gsem):
        wid = lax.axis_index("s") * info.num_cores + lax.axis_index("c")
        workers_per_phase = _FN // rows_per_worker
        phase = wid // workers_per_phase
        k0 = (wid % workers_per_phase) * rows_per_worker
        for c in range(rows_per_worker // chunk):
            for r in range(chunk // 16):
                kk = k0 + c * chunk + r * 16 + lax.iota(jnp.int32, 16)
                v = (_L - _NPH) + phase - kk
                v = jnp.where(v < 0, v + max_len, v)
                idx_v[pl.ds(r * 16, 16)] = v
            pltpu.async_copy(table_hbm.at[idx_v], rows_v, gsem).wait()
            pltpu.sync_copy(rows_v, f_hbm.at[phase, pl.ds(k0 + c * chunk, chunk)])

    return k(table)


def _fan_out_tc(f8):
    nph, fn, d_model = f8.shape

    def body(f_ref, o_hbm, sem):
        def issue(i, _):
            r = i % _NPH
            start = pl.multiple_of((_L - _NPH) - (i - r), _NPH)
            pltpu.make_async_copy(
                f_ref.at[r, pl.ds(start, _L), :], o_hbm.at[i], sem
            ).start()
            return 0

        lax.fori_loop(0, _L, issue, 0)

        def drain(i, _):
            pltpu.make_async_copy(
                f_ref.at[0, pl.ds(0, _L), :], o_hbm.at[0], sem
            ).wait()
            return 0

        lax.fori_loop(0, _L, drain, 0)

    return pl.pallas_call(
        body,
        in_specs=[pl.BlockSpec(memory_space=pltpu.VMEM)],
        out_specs=pl.BlockSpec(memory_space=pl.ANY),
        out_shape=jax.ShapeDtypeStruct((_L, _L, d_model), jnp.float32),
        scratch_shapes=[pltpu.SemaphoreType.DMA],
    )(f8)


def kernel(rel_pos_embed, length):
    del length
    return _fan_out_tc(_build_f8_sc(rel_pos_embed))


# speedup vs baseline: 1.0291x; 1.0291x over previous
import jax
import jax.numpy as jnp
from jax import lax
from jax.experimental import pallas as pl
from jax.experimental.pallas import tpu as pltpu
from jax.experimental.pallas import tpu_sc as plsc

_L = 512
_FN = 1024
_NPH = 8


def _build_f8_sc(table):
    max_len, d_model = table.shape

    info = plsc.get_sparse_core_info()
    num_workers = info.num_cores * info.num_subcores
    rows_per_worker = _NPH * _FN // num_workers
    chunk = 64
    nchunks = rows_per_worker // chunk

    mesh = plsc.VectorSubcoreMesh(core_axis_name="c", subcore_axis_name="s")

    @pl.kernel(
        out_type=jax.ShapeDtypeStruct((_NPH, _FN, d_model), jnp.float32),
        mesh=mesh,
        scratch_types=[
            pltpu.VMEM((2, chunk), jnp.int32),
            pltpu.VMEM((2, chunk, d_model), jnp.float32),
            pltpu.SemaphoreType.DMA,
            pltpu.SemaphoreType.DMA,
        ],
    )
    def k(table_hbm, f_hbm, idx_v, rows_v, gsem, wsem):
        wid = lax.axis_index("s") * info.num_cores + lax.axis_index("c")
        workers_per_phase = _FN // rows_per_worker
        phase = wid // workers_per_phase
        k0 = (wid % workers_per_phase) * rows_per_worker
        writes = []
        for c in range(nchunks):
            b = c % 2
            if c >= 2:
                writes[c - 2].wait()
            for r in range(chunk // 16):
                kk = k0 + c * chunk + r * 16 + lax.iota(jnp.int32, 16)
                v = (_L - _NPH) + phase - kk
                v = jnp.where(v < 0, v + max_len, v)
                idx_v[b, pl.ds(r * 16, 16)] = v
            pltpu.async_copy(table_hbm.at[idx_v.at[b]], rows_v.at[b], gsem).wait()
            writes.append(
                pltpu.async_copy(
                    rows_v.at[b], f_hbm.at[phase, pl.ds(k0 + c * chunk, chunk)], wsem
                )
            )
        for w in writes[-2:]:
            w.wait()

    return k(table)


def _fan_out_tc(f8):
    nph, fn, d_model = f8.shape

    def body(f_hbm, o_hbm, fv, lsem, osem):
        for r in range(_NPH):
            pltpu.make_async_copy(f_hbm.at[r], fv.at[r], lsem).start()
        for r in range(_NPH):
            pltpu.make_async_copy(f_hbm.at[r], fv.at[r], lsem).wait()

            def issue(q, _):
                start = pl.multiple_of((_L - _NPH) - _NPH * q, _NPH)
                pltpu.make_async_copy(
                    fv.at[r, pl.ds(start, _L), :], o_hbm.at[_NPH * q + r], osem
                ).start()
                return 0

            lax.fori_loop(0, _L // _NPH, issue, 0)

        def drain(i, _):
            pltpu.make_async_copy(
                fv.at[0, pl.ds(0, _L), :], o_hbm.at[0], osem
            ).wait()
            return 0

        lax.fori_loop(0, _L, drain, 0)

    return pl.pallas_call(
        body,
        in_specs=[pl.BlockSpec(memory_space=pl.ANY)],
        out_specs=pl.BlockSpec(memory_space=pl.ANY),
        out_shape=jax.ShapeDtypeStruct((_L, _L, d_model), jnp.float32),
        scratch_shapes=[
            pltpu.VMEM((nph, fn, d_model), jnp.float32),
            pltpu.SemaphoreType.DMA,
            pltpu.SemaphoreType.DMA,
        ],
    )(f8)


def kernel(rel_pos_embed, length):
    del length
    return _fan_out_tc(_build_f8_sc(rel_pos_embed))


# speedup vs baseline: 1.1031x; 1.0720x over previous
import jax
import jax.numpy as jnp
from jax import lax
from jax.experimental import pallas as pl
from jax.experimental.pallas import tpu as pltpu
from jax.experimental.pallas import tpu_sc as plsc

_L = 512
_FN = 1032
_NPH = 8


def _build_f_sc(table):
    max_len, d_model = table.shape

    info = plsc.get_sparse_core_info()
    num_workers = info.num_cores * info.num_subcores
    base_rows = 32
    gather_rows = 48

    mesh = plsc.VectorSubcoreMesh(core_axis_name="c", subcore_axis_name="s")

    @pl.kernel(
        out_type=jax.ShapeDtypeStruct((_FN, d_model), jnp.float32),
        mesh=mesh,
        scratch_types=[
            pltpu.VMEM((gather_rows,), jnp.int32),
            pltpu.VMEM((gather_rows, d_model), jnp.float32),
            pltpu.SemaphoreType.DMA,
        ],
    )
    def k(table_hbm, f_hbm, idx_v, rows_v, gsem):
        wid = lax.axis_index("s") * info.num_cores + lax.axis_index("c")
        base = wid * base_rows
        for r in range(gather_rows // 16):
            kk = base + r * 16 + lax.iota(jnp.int32, 16)
            v = (_L - 1) - kk
            v = jnp.where(v < 0, v + max_len, v)
            idx_v[pl.ds(r * 16, 16)] = v
        pltpu.async_copy(table_hbm.at[idx_v], rows_v, gsem).wait()
        pltpu.sync_copy(rows_v.at[pl.ds(0, base_rows)], f_hbm.at[pl.ds(base, base_rows)])

        @pl.when(wid == num_workers - 1)
        def _():
            pltpu.sync_copy(
                rows_v.at[pl.ds(base_rows, _NPH)],
                f_hbm.at[pl.ds(base + base_rows, _NPH)],
            )

    return k(table)


def _fan_out_tc(f):
    fn, d_model = f.shape
    fpad = _FN - _NPH

    def body(f_ref, o_hbm, f8v, osem):
        copies_per_phase = _L // _NPH

        for r in range(_NPH):
            b = r % 2
            if r >= 2:
                def drain(i, _):
                    pltpu.make_async_copy(
                        f8v.at[0, pl.ds(0, _L), :], o_hbm.at[0], osem
                    ).wait()
                    return 0

                lax.fori_loop(0, copies_per_phase, drain, 0)

            f8v[b] = f_ref[pl.ds(_NPH - 1 - r, fpad), :]

            def issue(q, _):
                start = pl.multiple_of((_L - _NPH) - _NPH * q, _NPH)
                pltpu.make_async_copy(
                    f8v.at[b, pl.ds(start, _L), :], o_hbm.at[_NPH * q + r], osem
                ).start()
                return 0

            lax.fori_loop(0, copies_per_phase, issue, 0)

        def drain_tail(i, _):
            pltpu.make_async_copy(
                f8v.at[0, pl.ds(0, _L), :], o_hbm.at[0], osem
            ).wait()
            return 0

        lax.fori_loop(0, 2 * copies_per_phase, drain_tail, 0)

    return pl.pallas_call(
        body,
        in_specs=[pl.BlockSpec(memory_space=pltpu.VMEM)],
        out_specs=pl.BlockSpec(memory_space=pl.ANY),
        out_shape=jax.ShapeDtypeStruct((_L, _L, d_model), jnp.float32),
        scratch_shapes=[
            pltpu.VMEM((2, fpad, d_model), jnp.float32),
            pltpu.SemaphoreType.DMA,
        ],
    )(f)


def kernel(rel_pos_embed, length):
    del length
    return _fan_out_tc(_build_f_sc(rel_pos_embed))


# speedup vs baseline: 1.1083x; 1.0047x over previous
import jax
import jax.numpy as jnp
from jax import lax
from jax.experimental import pallas as pl
from jax.experimental.pallas import tpu as pltpu
from jax.experimental.pallas import tpu_sc as plsc

_L = 512
_FN = 1032
_NPH = 8


def _build_f_sc(table):
    max_len, d_model = table.shape

    info = plsc.get_sparse_core_info()
    num_workers = info.num_cores * info.num_subcores
    base_rows = 32
    gather_rows = 48

    mesh = plsc.VectorSubcoreMesh(core_axis_name="c", subcore_axis_name="s")

    @pl.kernel(
        out_type=jax.ShapeDtypeStruct((_FN, d_model), jnp.float32),
        mesh=mesh,
        scratch_types=[
            pltpu.VMEM((gather_rows,), jnp.int32),
            pltpu.VMEM((gather_rows, d_model), jnp.float32),
            pltpu.SemaphoreType.DMA,
        ],
    )
    def k(table_hbm, f_hbm, idx_v, rows_v, gsem):
        wid = lax.axis_index("s") * info.num_cores + lax.axis_index("c")
        base = wid * base_rows
        for r in range(gather_rows // 16):
            kk = base + r * 16 + lax.iota(jnp.int32, 16)
            v = (_L - 1) - kk
            v = jnp.where(v < 0, v + max_len, v)
            idx_v[pl.ds(r * 16, 16)] = v
        pltpu.async_copy(table_hbm.at[idx_v], rows_v, gsem).wait()
        pltpu.sync_copy(rows_v.at[pl.ds(0, base_rows)], f_hbm.at[pl.ds(base, base_rows)])

        @pl.when(wid == num_workers - 1)
        def _():
            pltpu.sync_copy(
                rows_v.at[pl.ds(base_rows, _NPH)],
                f_hbm.at[pl.ds(base + base_rows, _NPH)],
            )

    return k(table)


def _fan_out_tc(f):
    d_model = f.shape[1]
    fpad = _FN - _NPH

    def body(f_ref, o_hbm, f8v, osem):
        copies_per_phase = _L // _NPH

        for r in range(_NPH):
            b = r % 2
            if r >= 2:
                def drain(i, _):
                    pltpu.make_async_copy(
                        f8v.at[0, pl.ds(0, _L), :], o_hbm.at[0], osem
                    ).wait()
                    return 0

                lax.fori_loop(0, copies_per_phase, drain, 0)

            f8v[b] = f_ref[pl.ds(_NPH - 1 - r, fpad), :]

            def issue(q, _):
                start = pl.multiple_of((_L - _NPH) - _NPH * q, _NPH)
                pltpu.make_async_copy(
                    f8v.at[b, pl.ds(start, _L), :], o_hbm.at[_NPH * q + r], osem
                ).start()
                return 0

            lax.fori_loop(0, copies_per_phase, issue, 0)

        def drain_tail(i, _):
            pltpu.make_async_copy(
                f8v.at[0, pl.ds(0, _L), :], o_hbm.at[0], osem
            ).wait()
            return 0

        lax.fori_loop(0, 2 * copies_per_phase, drain_tail, 0)

    return pl.pallas_call(
        body,
        in_specs=[pl.BlockSpec(memory_space=pltpu.VMEM)],
        out_specs=pl.BlockSpec(memory_space=pl.ANY),
        out_shape=jax.ShapeDtypeStruct((_L, _L, d_model), jnp.float32),
        scratch_shapes=[
            pltpu.VMEM((2, fpad, d_model), jnp.float32),
            pltpu.SemaphoreType.DMA,
        ],
    )(f)


def kernel(rel_pos_embed, length):
    del length
    return _fan_out_tc(_build_f_sc(rel_pos_embed))
